# baseline (speedup 1.0000x reference)
import jax
import jax.numpy as jnp
from jax import lax
from jax.experimental import pallas as pl
from jax.experimental.pallas import tpu as pltpu

NUM_EXPERTS = 64
TOP_K = 8
HIDDEN = 1024
ROW_TILE = 4096
HALF = ROW_TILE // 2


def _topk_half(logits_t):
    r = logits_t.shape[1]
    iota = lax.broadcasted_iota(jnp.int32, (NUM_EXPERTS, r), 0)
    work = logits_t
    vals = []
    idxs = []
    for _ in range(TOP_K):
        m = jnp.max(work, axis=0, keepdims=True)
        is_max = work == m
        idx = jnp.min(jnp.where(is_max, iota, NUM_EXPERTS), axis=0,
                      keepdims=True)
        vals.append(m)
        idxs.append(idx)
        work = jnp.where(iota == idx, -jnp.inf, work)
    topl = jnp.concatenate(vals, axis=0)
    topi = jnp.concatenate(idxs, axis=0)

    e = jnp.exp(topl - topl[0:1, :])
    topv = e / jnp.sum(e, axis=0, keepdims=True)
    return topv.T, topi.T


def _router_body(hs_a_ref, hs_b_ref, w_ref, logits_ref, topv_ref, topi_ref):
    w = w_ref[...]
    for h, hs_ref in enumerate((hs_a_ref, hs_b_ref)):
        hs = hs_ref[...]
        logits = jax.lax.dot_general(
            hs, w,
            dimension_numbers=(((1,), (1,)), ((), ())),
            preferred_element_type=jnp.float32,
        )
        logits_ref[pl.ds(h * HALF, HALF), :] = logits

        logits_t = jax.lax.dot_general(
            w, hs,
            dimension_numbers=(((1,), (1,)), ((), ())),
            preferred_element_type=jnp.float32,
        )
        topv, topi = _topk_half(logits_t)
        topv_ref[pl.ds(h * HALF, HALF), :] = topv
        topi_ref[pl.ds(h * HALF, HALF), :] = topi


def kernel(hidden_states, weight, interpret=False):
    hs = hidden_states.reshape(-1, HIDDEN)
    n_rows = hs.shape[0]
    grid = (n_rows // ROW_TILE,)
    logits, topv, topi = pl.pallas_call(
        _router_body,
        grid=grid,
        in_specs=[
            pl.BlockSpec((HALF, HIDDEN), lambda i: (2 * i, 0)),
            pl.BlockSpec((HALF, HIDDEN), lambda i: (2 * i + 1, 0)),
            pl.BlockSpec((NUM_EXPERTS, HIDDEN), lambda i: (0, 0)),
        ],
        out_specs=[
            pl.BlockSpec((ROW_TILE, NUM_EXPERTS), lambda i: (i, 0)),
            pl.BlockSpec((ROW_TILE, TOP_K), lambda i: (i, 0)),
            pl.BlockSpec((ROW_TILE, TOP_K), lambda i: (i, 0)),
        ],
        out_shape=[
            jax.ShapeDtypeStruct((n_rows, NUM_EXPERTS), jnp.float32),
            jax.ShapeDtypeStruct((n_rows, TOP_K), jnp.float32),
            jax.ShapeDtypeStruct((n_rows, TOP_K), jnp.int32),
        ],
        compiler_params=pltpu.CompilerParams(
            dimension_semantics=("arbitrary",),
        ),
        interpret=interpret,
    )(hs, hs, weight)
    return (logits, topv, topi)


# speedup vs baseline: 1.0021x; 1.0021x over previous
import jax
import jax.numpy as jnp
from jax import lax
from jax.experimental import pallas as pl
from jax.experimental.pallas import tpu as pltpu

NUM_EXPERTS = 64
TOP_K = 8
HIDDEN = 1024
ROW_TILE = 4096


def _router_body(hs_ref, w_ref, logits_ref, topv_ref, topi_ref):
    hs = hs_ref[...]
    w = w_ref[...]
    logits = jax.lax.dot_general(
        hs, w,
        dimension_numbers=(((1,), (1,)), ((), ())),
        preferred_element_type=jnp.float32,
    )
    logits_ref[...] = logits

    logits_t = jax.lax.dot_general(
        w, hs,
        dimension_numbers=(((1,), (1,)), ((), ())),
        preferred_element_type=jnp.float32,
    )

    r = logits.shape[0]
    iota = lax.broadcasted_iota(jnp.int32, (NUM_EXPERTS, r), 0)
    work = logits_t
    vals = []
    idxs = []
    for _ in range(TOP_K):
        m = jnp.max(work, axis=0, keepdims=True)
        is_max = work == m
        idx = jnp.min(jnp.where(is_max, iota, NUM_EXPERTS), axis=0,
                      keepdims=True)
        vals.append(m)
        idxs.append(idx)
        work = jnp.where(iota == idx, -jnp.inf, work)
    topl = jnp.concatenate(vals, axis=0)
    topi = jnp.concatenate(idxs, axis=0)

    e = jnp.exp(topl - topl[0:1, :])
    topv = e / jnp.sum(e, axis=0, keepdims=True)
    topv_ref[...] = topv.T
    topi_ref[...] = topi.T


def kernel(hidden_states, weight):
    hs = hidden_states.reshape(-1, HIDDEN)
    n_rows = hs.shape[0]
    grid = (n_rows // ROW_TILE,)
    logits, topv, topi = pl.pallas_call(
        _router_body,
        grid=grid,
        in_specs=[
            pl.BlockSpec((ROW_TILE, HIDDEN), lambda i: (i, 0)),
            pl.BlockSpec((NUM_EXPERTS, HIDDEN), lambda i: (0, 0)),
        ],
        out_specs=[
            pl.BlockSpec((ROW_TILE, NUM_EXPERTS), lambda i: (i, 0)),
            pl.BlockSpec((ROW_TILE, TOP_K), lambda i: (i, 0)),
            pl.BlockSpec((ROW_TILE, TOP_K), lambda i: (i, 0)),
        ],
        out_shape=[
            jax.ShapeDtypeStruct((n_rows, NUM_EXPERTS), jnp.float32),
            jax.ShapeDtypeStruct((n_rows, TOP_K), jnp.float32),
            jax.ShapeDtypeStruct((n_rows, TOP_K), jnp.int32),
        ],
        compiler_params=pltpu.CompilerParams(
            dimension_semantics=("arbitrary",),
        ),
    )(hs, weight)
    return (logits, topv, topi)
